# Initial kernel scaffold; baseline (speedup 1.0000x reference)
import jax
import jax.numpy as jnp
from jax.experimental import pallas as pl


def kernel(x, edge_index, batch, W1, b1, W2, b2):
    raise NotImplementedError("write your pallas kernel here")


# speedup vs baseline: 9.0715x; 9.0715x over previous
import functools

import jax
import jax.numpy as jnp
from jax import lax
from jax.experimental import pallas as pl
from jax.experimental.pallas import tpu as pltpu
from jax.experimental.pallas import tpu_sc as plsc

N = 10000
E = 160000
IN_DIM = 256
HID = 128
OUT_DIM = 64
NG = 16

NC, NS = 2, 16
NW = NC * NS
CH = 128
KCH = 40
EPAD = NW * KCH * CH
ROWS = 10240
RPT = ROWS // NS
USZ = ROWS * NG
UPT = USZ // NS
ZCH = 2048

TM = 1280
GRID = ROWS // TM

_sc_mesh = plsc.VectorSubcoreMesh(core_axis_name="c", subcore_axis_name="s")
_sc_params = pltpu.CompilerParams(needs_layout_passes=False)


@functools.partial(
    pl.kernel,
    out_type=jax.ShapeDtypeStruct((NC, ROWS), jnp.float32),
    mesh=_sc_mesh,
    scratch_types=[
        pltpu.VMEM((CH,), jnp.int32),
        pltpu.VMEM((CH,), jnp.float32),
        pltpu.VMEM_SHARED((ROWS,), jnp.float32),
    ],
)
def _sc_degree(dst_hbm, ones_hbm, zeros_hbm, deg_out, idx_v, ones_v, deg_sh):
    cid = lax.axis_index("c")
    sid = lax.axis_index("s")
    w = cid * NS + sid
    pltpu.sync_copy(zeros_hbm.at[pl.ds(0, RPT)], deg_sh.at[pl.ds(sid * RPT, RPT)])
    pltpu.sync_copy(ones_hbm, ones_v)
    plsc.subcore_barrier()

    def body(j, carry):
        pltpu.sync_copy(dst_hbm.at[w, j], idx_v)
        pltpu.sync_copy(ones_v, deg_sh.at[idx_v], add=True)
        return carry

    lax.fori_loop(0, KCH, body, 0)
    plsc.subcore_barrier()
    pltpu.sync_copy(deg_sh.at[pl.ds(sid * RPT, RPT)],
                    deg_out.at[cid, pl.ds(sid * RPT, RPT)])


@functools.partial(
    pl.kernel,
    out_type=(jax.ShapeDtypeStruct((NC, ROWS, HID), jnp.float32),
              jax.ShapeDtypeStruct((NC, USZ), jnp.float32)),
    mesh=_sc_mesh,
    scratch_types=[
        pltpu.VMEM((CH,), jnp.int32),
        pltpu.VMEM((CH,), jnp.int32),
        pltpu.VMEM((CH, HID), jnp.float32),
        pltpu.VMEM((ROWS,), jnp.float32),
        pltpu.VMEM((ROWS,), jnp.int32),
        pltpu.VMEM((CH,), jnp.float32),
        pltpu.VMEM((CH,), jnp.int32),
        pltpu.VMEM_SHARED((ROWS, HID), jnp.float32),
        pltpu.VMEM_SHARED((USZ,), jnp.float32),
    ],
    compiler_params=_sc_params,
)
def _sc_edges(src_hbm, dst_hbm, g_hbm, dinv_hbm, batch_hbm, z2d_hbm, z1d_hbm,
              acc_out, u_out,
              src_v, dst_v, rows_v, dinv_v, batch_v, uval_v, uidx_v,
              acc_sh, u_sh):
    cid = lax.axis_index("c")
    sid = lax.axis_index("s")
    w = cid * NS + sid

    def zacc(k, carry):
        pltpu.sync_copy(z2d_hbm, acc_sh.at[pl.ds(sid * RPT + k * 16, 16)])
        return carry

    lax.fori_loop(0, RPT // 16, zacc, 0)

    def zu(k, carry):
        pltpu.sync_copy(z1d_hbm, u_sh.at[pl.ds(sid * UPT + k * ZCH, ZCH)])
        return carry

    lax.fori_loop(0, UPT // ZCH, zu, 0)

    pltpu.sync_copy(dinv_hbm, dinv_v)
    pltpu.sync_copy(batch_hbm, batch_v)
    plsc.subcore_barrier()

    def chunk(j, carry):
        pltpu.sync_copy(src_hbm.at[w, j], src_v)
        pltpu.sync_copy(dst_hbm.at[w, j], dst_v)
        pltpu.sync_copy(g_hbm.at[src_v], rows_v)
        pltpu.sync_copy(rows_v, acc_sh.at[dst_v], add=True)
        for i in range(CH // 16):
            s16 = src_v[pl.ds(i * 16, 16)]
            d16 = dst_v[pl.ds(i * 16, 16)]
            dv_s = plsc.load_gather(dinv_v, [s16])
            dv_d = plsc.load_gather(dinv_v, [d16])
            b_d = plsc.load_gather(batch_v, [d16])
            uval_v[pl.ds(i * 16, 16)] = dv_s * dv_d
            uidx_v[pl.ds(i * 16, 16)] = s16 * NG + b_d
        pltpu.sync_copy(uval_v, u_sh.at[uidx_v], add=True)
        return carry

    lax.fori_loop(0, KCH, chunk, 0)
    plsc.subcore_barrier()
    pltpu.sync_copy(acc_sh.at[pl.ds(sid * RPT, RPT)],
                    acc_out.at[cid, pl.ds(sid * RPT, RPT)])
    pltpu.sync_copy(u_sh.at[pl.ds(sid * UPT, UPT)],
                    u_out.at[cid, pl.ds(sid * UPT, UPT)])


def _mm_body(x_ref, w_ref, d0_ref, d1_ref, g_ref, dinv_ref):
    i = pl.program_id(0)
    h1 = jnp.dot(x_ref[...], w_ref[...], preferred_element_type=jnp.float32)
    deg = d0_ref[...] + d1_ref[...] + 1.0
    rid = i * TM + lax.broadcasted_iota(jnp.int32, (TM, 1), 0)
    dinv = jnp.where(rid < N, lax.rsqrt(deg), 0.0)
    g_ref[...] = h1 * dinv
    dinv_ref[...] = dinv


def _tc_matmul(xp, W1, deg0, deg1):
    return pl.pallas_call(
        _mm_body,
        grid=(GRID,),
        in_specs=[
            pl.BlockSpec((TM, IN_DIM), lambda i: (i, 0)),
            pl.BlockSpec((IN_DIM, HID), lambda i: (0, 0)),
            pl.BlockSpec((TM, 1), lambda i: (i, 0)),
            pl.BlockSpec((TM, 1), lambda i: (i, 0)),
        ],
        out_specs=[
            pl.BlockSpec((TM, HID), lambda i: (i, 0)),
            pl.BlockSpec((TM, 1), lambda i: (i, 0)),
        ],
        out_shape=[
            jax.ShapeDtypeStruct((ROWS, HID), jnp.float32),
            jax.ShapeDtypeStruct((ROWS, 1), jnp.float32),
        ],
    )(xp, W1, deg0, deg1)


def _fin_body(a0_ref, a1_ref, g_ref, dv_ref, u0_ref, u1_ref, bt_ref,
              b1_ref, w2_ref, b2_ref, out_ref, p_acc, c_acc):
    k = pl.program_id(0)

    @pl.when(k == 0)
    def _():
        p_acc[...] = jnp.zeros_like(p_acc)
        c_acc[...] = jnp.zeros_like(c_acc)

    dv = dv_ref[...]
    h = jnp.maximum(dv * (a0_ref[...] + a1_ref[...] + g_ref[...]) + b1_ref[...],
                    0.0)
    gids = lax.broadcasted_iota(jnp.int32, (TM, NG), 1)
    onehot = (bt_ref[...] == gids).astype(jnp.float32)
    ueff = u0_ref[...] + u1_ref[...] + onehot * (dv * dv)
    dn = (((0,), (0,)), ((), ()))
    p_acc[...] += lax.dot_general(ueff, h, dn, preferred_element_type=jnp.float32)
    c_acc[...] += lax.dot_general(onehot, jnp.ones((TM, HID), jnp.float32), dn,
                                  preferred_element_type=jnp.float32)

    @pl.when(k == GRID - 1)
    def _():
        cnt = jnp.maximum(c_acc[...], 1.0)
        out_ref[...] = (jnp.dot(p_acc[...] / cnt, w2_ref[...],
                                preferred_element_type=jnp.float32)
                        + b2_ref[...])


def _tc_final(acc0, acc1, g_pad, dinv_col, u0, u1, bt, b1r, W2, b2r):
    return pl.pallas_call(
        _fin_body,
        grid=(GRID,),
        in_specs=[
            pl.BlockSpec((TM, HID), lambda k: (k, 0)),
            pl.BlockSpec((TM, HID), lambda k: (k, 0)),
            pl.BlockSpec((TM, HID), lambda k: (k, 0)),
            pl.BlockSpec((TM, 1), lambda k: (k, 0)),
            pl.BlockSpec((TM, NG), lambda k: (k, 0)),
            pl.BlockSpec((TM, NG), lambda k: (k, 0)),
            pl.BlockSpec((TM, 1), lambda k: (k, 0)),
            pl.BlockSpec((1, HID), lambda k: (0, 0)),
            pl.BlockSpec((HID, OUT_DIM), lambda k: (0, 0)),
            pl.BlockSpec((1, OUT_DIM), lambda k: (0, 0)),
        ],
        out_specs=pl.BlockSpec((NG, OUT_DIM), lambda k: (0, 0)),
        out_shape=jax.ShapeDtypeStruct((NG, OUT_DIM), jnp.float32),
        scratch_shapes=[
            pltpu.VMEM((NG, HID), jnp.float32),
            pltpu.VMEM((NG, HID), jnp.float32),
        ],
    )(acc0, acc1, g_pad, dinv_col, u0, u1, bt, b1r, W2, b2r)


def kernel(x, edge_index, batch, W1, b1, W2, b2):
    x = x.astype(jnp.float32)
    src = edge_index[0]
    dst = edge_index[1]
    pad_e = jnp.full((EPAD - E,), N, jnp.int32)
    srcp = jnp.concatenate([src, pad_e]).reshape(NW, KCH, CH)
    dstp = jnp.concatenate([dst, pad_e]).reshape(NW, KCH, CH)
    batchp = jnp.concatenate([batch, jnp.full((ROWS - N,), NG, jnp.int32)])
    xp = jnp.pad(x, ((0, ROWS - N), (0, 0)))
    zeros1d = jnp.zeros((ZCH,), jnp.float32)
    zeros2d = jnp.zeros((16, HID), jnp.float32)
    ones128 = jnp.ones((CH,), jnp.float32)

    deg2 = _sc_degree(dstp, ones128, zeros1d)
    deg0 = deg2[0].reshape(ROWS, 1)
    deg1 = deg2[1].reshape(ROWS, 1)

    g_pad, dinv_col = _tc_matmul(xp, W1, deg0, deg1)

    acc2, u2 = _sc_edges(srcp, dstp, g_pad, dinv_col.reshape(ROWS), batchp,
                         zeros2d, zeros1d)

    u0 = u2[0].reshape(ROWS, NG)
    u1 = u2[1].reshape(ROWS, NG)
    pool = _tc_final(acc2[0], acc2[1], g_pad, dinv_col, u0, u1,
                     batchp.reshape(ROWS, 1), b1.reshape(1, HID), W2,
                     b2.reshape(1, OUT_DIM))
    return pool


# speedup vs baseline: 11.9842x; 1.3211x over previous
import functools

import jax
import jax.numpy as jnp
from jax import lax
from jax.experimental import pallas as pl
from jax.experimental.pallas import tpu as pltpu
from jax.experimental.pallas import tpu_sc as plsc

N = 10000
E = 160000
IN_DIM = 256
HID = 128
OUT_DIM = 64
NG = 16

NC, NS = 2, 16
NW = NC * NS
CH = 80
KCH = 64
EPAD = NW * KCH * CH
ROWS = 10240
RPT = ROWS // NS
USZ = ROWS * NG
UPT = USZ // NS
ZCH = 2048

TM = 1280
GRID = ROWS // TM

_sc_mesh = plsc.VectorSubcoreMesh(core_axis_name="c", subcore_axis_name="s")
_sc_params = pltpu.CompilerParams(needs_layout_passes=False)


@functools.partial(
    pl.kernel,
    out_type=jax.ShapeDtypeStruct((NC, ROWS), jnp.float32),
    mesh=_sc_mesh,
    scratch_types=[
        pltpu.VMEM((KCH, CH), jnp.int32),
        pltpu.VMEM((CH,), jnp.float32),
        pltpu.VMEM_SHARED((ROWS,), jnp.float32),
    ],
)
def _sc_degree(dst_hbm, ones_hbm, zeros_hbm, deg_out, idx_v, ones_v, deg_sh):
    cid = lax.axis_index("c")
    sid = lax.axis_index("s")
    w = cid * NS + sid
    pltpu.sync_copy(zeros_hbm.at[pl.ds(0, RPT)], deg_sh.at[pl.ds(sid * RPT, RPT)])
    pltpu.sync_copy(ones_hbm, ones_v)
    pltpu.sync_copy(dst_hbm.at[w], idx_v)
    plsc.subcore_barrier()

    def body(j, carry):
        pltpu.sync_copy(ones_v, deg_sh.at[idx_v.at[j]], add=True)
        return carry

    lax.fori_loop(0, KCH, body, 0)
    plsc.subcore_barrier()
    pltpu.sync_copy(deg_sh.at[pl.ds(sid * RPT, RPT)],
                    deg_out.at[cid, pl.ds(sid * RPT, RPT)])


@functools.partial(
    pl.kernel,
    out_type=(jax.ShapeDtypeStruct((NC, ROWS, HID), jnp.float32),
              jax.ShapeDtypeStruct((NC, USZ), jnp.float32)),
    mesh=_sc_mesh,
    scratch_types=[
        pltpu.VMEM((KCH, CH), jnp.int32),
        pltpu.VMEM((KCH, CH), jnp.int32),
        pltpu.VMEM((CH, HID), jnp.float32),
        pltpu.VMEM((CH, HID), jnp.float32),
        pltpu.VMEM((CH,), jnp.float32),
        pltpu.VMEM((CH,), jnp.float32),
        pltpu.VMEM((CH,), jnp.int32),
        pltpu.VMEM((CH,), jnp.int32),
        pltpu.VMEM((CH,), jnp.int32),
        pltpu.VMEM((CH,), jnp.int32),
        pltpu.VMEM_SHARED((ROWS, HID), jnp.float32),
        pltpu.VMEM_SHARED((USZ,), jnp.float32),
        pltpu.SemaphoreType.DMA,
        pltpu.SemaphoreType.DMA,
        pltpu.SemaphoreType.DMA,
        pltpu.SemaphoreType.DMA,
    ],
    compiler_params=_sc_params,
)
def _sc_edges(src_hbm, dst_hbm, g_hbm, dinv_hbm, batch_hbm, z2d_hbm, z1d_hbm,
              acc_out, u_out,
              src_v, dst_v, rows0_v, rows1_v, dd0_v, dd1_v, bd0_v, bd1_v,
              uidx0_v, uidx1_v, acc_sh, u_sh, gsem0, gsem1, usem0, usem1):
    cid = lax.axis_index("c")
    sid = lax.axis_index("s")
    w = cid * NS + sid

    def zacc(k, carry):
        pltpu.sync_copy(z2d_hbm, acc_sh.at[pl.ds(sid * RPT + k * 16, 16)])
        return carry

    lax.fori_loop(0, RPT // 16, zacc, 0)

    def zu(k, carry):
        pltpu.sync_copy(z1d_hbm, u_sh.at[pl.ds(sid * UPT + k * ZCH, ZCH)])
        return carry

    lax.fori_loop(0, UPT // ZCH, zu, 0)

    pltpu.sync_copy(src_hbm.at[w], src_v)
    pltpu.sync_copy(dst_hbm.at[w], dst_v)
    plsc.subcore_barrier()

    def issue(j, rows_v, dd_v, bd_v, gsem, usem):
        pltpu.async_copy(g_hbm.at[src_v.at[j]], rows_v, gsem)
        pltpu.async_copy(dinv_hbm.at[dst_v.at[j]], dd_v, usem)
        pltpu.async_copy(batch_hbm.at[dst_v.at[j]], bd_v, usem)

    def process(j, rows_v, dd_v, bd_v, uidx_v, gsem, usem):
        pltpu.make_async_copy(g_hbm.at[src_v.at[j]], rows_v, gsem).wait()
        pltpu.sync_copy(rows_v, acc_sh.at[dst_v.at[j]], add=True)
        pltpu.make_async_copy(dinv_hbm.at[dst_v.at[j]], dd_v, usem).wait()
        pltpu.make_async_copy(batch_hbm.at[dst_v.at[j]], bd_v, usem).wait()
        for i in range(CH // 16):
            s16 = src_v[j, pl.ds(i * 16, 16)]
            b16 = bd_v[pl.ds(i * 16, 16)]
            uidx_v[pl.ds(i * 16, 16)] = s16 * NG + b16
        pltpu.sync_copy(dd_v, u_sh.at[uidx_v], add=True)

    issue(0, rows0_v, dd0_v, bd0_v, gsem0, usem0)

    def body2(t, carry):
        j0 = 2 * t
        j1 = j0 + 1
        issue(j1, rows1_v, dd1_v, bd1_v, gsem1, usem1)
        process(j0, rows0_v, dd0_v, bd0_v, uidx0_v, gsem0, usem0)

        @pl.when(t < KCH // 2 - 1)
        def _():
            issue(j0 + 2, rows0_v, dd0_v, bd0_v, gsem0, usem0)

        process(j1, rows1_v, dd1_v, bd1_v, uidx1_v, gsem1, usem1)
        return carry

    lax.fori_loop(0, KCH // 2, body2, 0)
    plsc.subcore_barrier()
    pltpu.sync_copy(acc_sh.at[pl.ds(sid * RPT, RPT)],
                    acc_out.at[cid, pl.ds(sid * RPT, RPT)])
    pltpu.sync_copy(u_sh.at[pl.ds(sid * UPT, UPT)],
                    u_out.at[cid, pl.ds(sid * UPT, UPT)])


def _mm_body(x_ref, w_ref, d0_ref, d1_ref, g_ref, dinv_ref):
    i = pl.program_id(0)
    h1 = jnp.dot(x_ref[...], w_ref[...], preferred_element_type=jnp.float32)
    deg = d0_ref[...] + d1_ref[...] + 1.0
    rid = i * TM + lax.broadcasted_iota(jnp.int32, (TM, 1), 0)
    dinv = jnp.where(rid < N, lax.rsqrt(deg), 0.0)
    g_ref[...] = h1 * dinv
    dinv_ref[...] = dinv


def _tc_matmul(xp, W1, deg0, deg1):
    return pl.pallas_call(
        _mm_body,
        grid=(GRID,),
        in_specs=[
            pl.BlockSpec((TM, IN_DIM), lambda i: (i, 0)),
            pl.BlockSpec((IN_DIM, HID), lambda i: (0, 0)),
            pl.BlockSpec((TM, 1), lambda i: (i, 0)),
            pl.BlockSpec((TM, 1), lambda i: (i, 0)),
        ],
        out_specs=[
            pl.BlockSpec((TM, HID), lambda i: (i, 0)),
            pl.BlockSpec((TM, 1), lambda i: (i, 0)),
        ],
        out_shape=[
            jax.ShapeDtypeStruct((ROWS, HID), jnp.float32),
            jax.ShapeDtypeStruct((ROWS, 1), jnp.float32),
        ],
    )(xp, W1, deg0, deg1)


def _fin_body(a0_ref, a1_ref, g_ref, dv_ref, u0_ref, u1_ref, bt_ref,
              b1_ref, w2_ref, b2_ref, out_ref, p_acc, c_acc):
    k = pl.program_id(0)

    @pl.when(k == 0)
    def _():
        p_acc[...] = jnp.zeros_like(p_acc)
        c_acc[...] = jnp.zeros_like(c_acc)

    dv = dv_ref[...]
    h = jnp.maximum(dv * (a0_ref[...] + a1_ref[...] + g_ref[...]) + b1_ref[...],
                    0.0)
    gids = lax.broadcasted_iota(jnp.int32, (TM, NG), 1)
    onehot = (bt_ref[...] == gids).astype(jnp.float32)
    ueff = (u0_ref[...] + u1_ref[...] + onehot * dv) * dv
    dn = (((0,), (0,)), ((), ()))
    p_acc[...] += lax.dot_general(ueff, h, dn, preferred_element_type=jnp.float32)
    c_acc[...] += lax.dot_general(onehot, jnp.ones((TM, HID), jnp.float32), dn,
                                  preferred_element_type=jnp.float32)

    @pl.when(k == GRID - 1)
    def _():
        cnt = jnp.maximum(c_acc[...], 1.0)
        out_ref[...] = (jnp.dot(p_acc[...] / cnt, w2_ref[...],
                                preferred_element_type=jnp.float32)
                        + b2_ref[...])


def _tc_final(acc0, acc1, g_pad, dinv_col, u0, u1, bt, b1r, W2, b2r):
    return pl.pallas_call(
        _fin_body,
        grid=(GRID,),
        in_specs=[
            pl.BlockSpec((TM, HID), lambda k: (k, 0)),
            pl.BlockSpec((TM, HID), lambda k: (k, 0)),
            pl.BlockSpec((TM, HID), lambda k: (k, 0)),
            pl.BlockSpec((TM, 1), lambda k: (k, 0)),
            pl.BlockSpec((TM, NG), lambda k: (k, 0)),
            pl.BlockSpec((TM, NG), lambda k: (k, 0)),
            pl.BlockSpec((TM, 1), lambda k: (k, 0)),
            pl.BlockSpec((1, HID), lambda k: (0, 0)),
            pl.BlockSpec((HID, OUT_DIM), lambda k: (0, 0)),
            pl.BlockSpec((1, OUT_DIM), lambda k: (0, 0)),
        ],
        out_specs=pl.BlockSpec((NG, OUT_DIM), lambda k: (0, 0)),
        out_shape=jax.ShapeDtypeStruct((NG, OUT_DIM), jnp.float32),
        scratch_shapes=[
            pltpu.VMEM((NG, HID), jnp.float32),
            pltpu.VMEM((NG, HID), jnp.float32),
        ],
    )(acc0, acc1, g_pad, dinv_col, u0, u1, bt, b1r, W2, b2r)


def kernel(x, edge_index, batch, W1, b1, W2, b2):
    x = x.astype(jnp.float32)
    src = edge_index[0]
    dst = edge_index[1]
    pad_e = jnp.full((EPAD - E,), N, jnp.int32)
    srcp = jnp.concatenate([src, pad_e]).reshape(NW, KCH, CH)
    dstp = jnp.concatenate([dst, pad_e]).reshape(NW, KCH, CH)
    batchp = jnp.concatenate([batch, jnp.full((ROWS - N,), NG, jnp.int32)])
    xp = jnp.pad(x, ((0, ROWS - N), (0, 0)))
    zeros1d = jnp.zeros((ZCH,), jnp.float32)
    zeros2d = jnp.zeros((16, HID), jnp.float32)
    ones128 = jnp.ones((CH,), jnp.float32)

    deg2 = _sc_degree(dstp, ones128, zeros1d)
    deg0 = deg2[0].reshape(ROWS, 1)
    deg1 = deg2[1].reshape(ROWS, 1)

    g_pad, dinv_col = _tc_matmul(xp, W1, deg0, deg1)

    acc2, u2 = _sc_edges(srcp, dstp, g_pad, dinv_col.reshape(ROWS), batchp,
                         zeros2d, zeros1d)

    u0 = u2[0].reshape(ROWS, NG)
    u1 = u2[1].reshape(ROWS, NG)
    pool = _tc_final(acc2[0], acc2[1], g_pad, dinv_col, u0, u1,
                     batchp.reshape(ROWS, 1), b1.reshape(1, HID), W2,
                     b2.reshape(1, OUT_DIM))
    return pool


# speedup vs baseline: 13.8044x; 1.1519x over previous
import functools

import jax
import jax.numpy as jnp
from jax import lax
from jax.experimental import pallas as pl
from jax.experimental.pallas import tpu as pltpu
from jax.experimental.pallas import tpu_sc as plsc

N = 10000
E = 160000
IN_DIM = 256
HID = 128
OUT_DIM = 64
NG = 16

NC, NS = 2, 16
NW = NC * NS
CH = 80
KCH = 64
EPAD = NW * KCH * CH
ROWS = 10240
RPT = ROWS // NS
USZ = ROWS * NG
UPT = USZ // NS
ZCH = 512

TM = 1280
GRID = ROWS // TM

_sc_mesh = plsc.VectorSubcoreMesh(core_axis_name="c", subcore_axis_name="s")
_sc_params = pltpu.CompilerParams(needs_layout_passes=False)


@functools.partial(
    pl.kernel,
    out_type=jax.ShapeDtypeStruct((NC, ROWS), jnp.float32),
    mesh=_sc_mesh,
    scratch_types=[
        pltpu.VMEM((KCH, CH), jnp.int32),
        pltpu.VMEM((CH,), jnp.float32),
        pltpu.VMEM_SHARED((ROWS,), jnp.float32),
    ],
)
def _sc_degree(dst_hbm, ones_hbm, zeros_hbm, deg_out, idx_v, ones_v, deg_sh):
    cid = lax.axis_index("c")
    sid = lax.axis_index("s")
    w = cid * NS + sid
    pltpu.sync_copy(zeros_hbm.at[pl.ds(0, RPT)], deg_sh.at[pl.ds(sid * RPT, RPT)])
    pltpu.sync_copy(ones_hbm, ones_v)
    pltpu.sync_copy(dst_hbm.at[w], idx_v)
    plsc.subcore_barrier()

    def body(j, carry):
        pltpu.sync_copy(ones_v, deg_sh.at[idx_v.at[j]], add=True)
        return carry

    lax.fori_loop(0, KCH, body, 0)
    plsc.subcore_barrier()
    pltpu.sync_copy(deg_sh.at[pl.ds(sid * RPT, RPT)],
                    deg_out.at[cid, pl.ds(sid * RPT, RPT)])


@functools.partial(
    pl.kernel,
    out_type=(jax.ShapeDtypeStruct((NC, ROWS, HID), jnp.float32),
              jax.ShapeDtypeStruct((NC, USZ), jnp.float32)),
    mesh=_sc_mesh,
    scratch_types=[
        pltpu.VMEM((KCH, CH), jnp.int32),
        pltpu.VMEM((KCH, CH), jnp.int32),
        pltpu.VMEM((CH, HID), jnp.float32),
        pltpu.VMEM((CH, HID), jnp.float32),
        pltpu.VMEM((CH,), jnp.float32),
        pltpu.VMEM((CH,), jnp.float32),
        pltpu.VMEM((CH,), jnp.int32),
        pltpu.VMEM((CH,), jnp.int32),
        pltpu.VMEM((CH,), jnp.int32),
        pltpu.VMEM((CH,), jnp.int32),
        pltpu.VMEM((ZCH,), jnp.float32),
        pltpu.VMEM_SHARED((ROWS, HID), jnp.float32),
        pltpu.VMEM_SHARED((USZ,), jnp.float32),
        pltpu.SemaphoreType.DMA,
        pltpu.SemaphoreType.DMA,
        pltpu.SemaphoreType.DMA,
        pltpu.SemaphoreType.DMA,
    ],
    compiler_params=_sc_params,
)
def _sc_edges(src_hbm, dst_hbm, g_hbm, dinv_hbm, batch_hbm,
              acc_out, u_out,
              src_v, dst_v, rows0_v, rows1_v, dd0_v, dd1_v, bd0_v, bd1_v,
              uidx0_v, uidx1_v, zbuf_v, acc_sh, u_sh,
              gsem0, gsem1, usem0, usem1):
    cid = lax.axis_index("c")
    sid = lax.axis_index("s")
    w = cid * NS + sid

    zv = jnp.zeros((16,), jnp.float32)

    def zrow(r, carry):
        for i in range(HID // 16):
            rows0_v[r, pl.ds(i * 16, 16)] = zv
        return carry

    lax.fori_loop(0, CH, zrow, 0)

    def zflat(k, carry):
        zbuf_v[pl.ds(k * 16, 16)] = zv
        return carry

    lax.fori_loop(0, ZCH // 16, zflat, 0)

    for k in range(RPT // CH):
        pltpu.async_copy(rows0_v, acc_sh.at[pl.ds(sid * RPT + k * CH, CH)], gsem0)
    for k in range(UPT // ZCH):
        pltpu.async_copy(zbuf_v, u_sh.at[pl.ds(sid * UPT + k * ZCH, ZCH)], gsem1)
    pltpu.async_copy(src_hbm.at[w], src_v, usem0)
    pltpu.async_copy(dst_hbm.at[w], dst_v, usem1)
    for k in range(RPT // CH):
        pltpu.make_async_copy(
            rows0_v, acc_sh.at[pl.ds(sid * RPT + k * CH, CH)], gsem0).wait()
    for k in range(UPT // ZCH):
        pltpu.make_async_copy(
            zbuf_v, u_sh.at[pl.ds(sid * UPT + k * ZCH, ZCH)], gsem1).wait()
    pltpu.make_async_copy(src_hbm.at[w], src_v, usem0).wait()
    pltpu.make_async_copy(dst_hbm.at[w], dst_v, usem1).wait()
    plsc.subcore_barrier()

    def issue(j, rows_v, dd_v, bd_v, gsem, usem):
        pltpu.async_copy(g_hbm.at[src_v.at[j]], rows_v, gsem)
        pltpu.async_copy(dinv_hbm.at[dst_v.at[j]], dd_v, usem)
        pltpu.async_copy(batch_hbm.at[dst_v.at[j]], bd_v, usem)

    def process(j, rows_v, dd_v, bd_v, uidx_v, gsem, usem):
        pltpu.make_async_copy(g_hbm.at[src_v.at[j]], rows_v, gsem).wait()
        pltpu.sync_copy(rows_v, acc_sh.at[dst_v.at[j]], add=True)
        pltpu.make_async_copy(dinv_hbm.at[dst_v.at[j]], dd_v, usem).wait()
        pltpu.make_async_copy(batch_hbm.at[dst_v.at[j]], bd_v, usem).wait()
        for i in range(CH // 16):
            s16 = src_v[j, pl.ds(i * 16, 16)]
            b16 = bd_v[pl.ds(i * 16, 16)]
            uidx_v[pl.ds(i * 16, 16)] = s16 * NG + b16
        pltpu.sync_copy(dd_v, u_sh.at[uidx_v], add=True)

    issue(0, rows0_v, dd0_v, bd0_v, gsem0, usem0)

    def body2(t, carry):
        j0 = 2 * t
        j1 = j0 + 1
        issue(j1, rows1_v, dd1_v, bd1_v, gsem1, usem1)
        process(j0, rows0_v, dd0_v, bd0_v, uidx0_v, gsem0, usem0)

        @pl.when(t < KCH // 2 - 1)
        def _():
            issue(j0 + 2, rows0_v, dd0_v, bd0_v, gsem0, usem0)

        process(j1, rows1_v, dd1_v, bd1_v, uidx1_v, gsem1, usem1)
        return carry

    lax.fori_loop(0, KCH // 2, body2, 0)
    plsc.subcore_barrier()
    pltpu.async_copy(acc_sh.at[pl.ds(sid * RPT, RPT)],
                     acc_out.at[cid, pl.ds(sid * RPT, RPT)], gsem0)
    pltpu.async_copy(u_sh.at[pl.ds(sid * UPT, UPT)],
                     u_out.at[cid, pl.ds(sid * UPT, UPT)], gsem1)
    pltpu.make_async_copy(acc_sh.at[pl.ds(sid * RPT, RPT)],
                          acc_out.at[cid, pl.ds(sid * RPT, RPT)], gsem0).wait()
    pltpu.make_async_copy(u_sh.at[pl.ds(sid * UPT, UPT)],
                          u_out.at[cid, pl.ds(sid * UPT, UPT)], gsem1).wait()


def _mm_body(x_ref, w_ref, d0_ref, d1_ref, g_ref, dinv_ref):
    i = pl.program_id(0)
    h1 = jnp.dot(x_ref[...], w_ref[...], preferred_element_type=jnp.float32)
    deg = d0_ref[...] + d1_ref[...] + 1.0
    rid = i * TM + lax.broadcasted_iota(jnp.int32, (TM, 1), 0)
    dinv = jnp.where(rid < N, lax.rsqrt(deg), 0.0)
    g_ref[...] = h1 * dinv
    dinv_ref[...] = dinv


def _tc_matmul(xp, W1, deg0, deg1):
    return pl.pallas_call(
        _mm_body,
        grid=(GRID,),
        in_specs=[
            pl.BlockSpec((TM, IN_DIM), lambda i: (i, 0)),
            pl.BlockSpec((IN_DIM, HID), lambda i: (0, 0)),
            pl.BlockSpec((TM, 1), lambda i: (i, 0)),
            pl.BlockSpec((TM, 1), lambda i: (i, 0)),
        ],
        out_specs=[
            pl.BlockSpec((TM, HID), lambda i: (i, 0)),
            pl.BlockSpec((TM, 1), lambda i: (i, 0)),
        ],
        out_shape=[
            jax.ShapeDtypeStruct((ROWS, HID), jnp.float32),
            jax.ShapeDtypeStruct((ROWS, 1), jnp.float32),
        ],
    )(xp, W1, deg0, deg1)


def _fin_body(a0_ref, a1_ref, g_ref, dv_ref, u0_ref, u1_ref, bt_ref,
              b1_ref, w2_ref, b2_ref, out_ref, p_acc, c_acc):
    k = pl.program_id(0)

    @pl.when(k == 0)
    def _():
        p_acc[...] = jnp.zeros_like(p_acc)
        c_acc[...] = jnp.zeros_like(c_acc)

    dv = dv_ref[...]
    h = jnp.maximum(dv * (a0_ref[...] + a1_ref[...] + g_ref[...]) + b1_ref[...],
                    0.0)
    gids = lax.broadcasted_iota(jnp.int32, (TM, NG), 1)
    onehot = (bt_ref[...] == gids).astype(jnp.float32)
    ueff = (u0_ref[...] + u1_ref[...] + onehot * dv) * dv
    dn = (((0,), (0,)), ((), ()))
    p_acc[...] += lax.dot_general(ueff, h, dn, preferred_element_type=jnp.float32)
    c_acc[...] += lax.dot_general(onehot, jnp.ones((TM, HID), jnp.float32), dn,
                                  preferred_element_type=jnp.float32)

    @pl.when(k == GRID - 1)
    def _():
        cnt = jnp.maximum(c_acc[...], 1.0)
        out_ref[...] = (jnp.dot(p_acc[...] / cnt, w2_ref[...],
                                preferred_element_type=jnp.float32)
                        + b2_ref[...])


def _tc_final(acc0, acc1, g_pad, dinv_col, u0, u1, bt, b1r, W2, b2r):
    return pl.pallas_call(
        _fin_body,
        grid=(GRID,),
        in_specs=[
            pl.BlockSpec((TM, HID), lambda k: (k, 0)),
            pl.BlockSpec((TM, HID), lambda k: (k, 0)),
            pl.BlockSpec((TM, HID), lambda k: (k, 0)),
            pl.BlockSpec((TM, 1), lambda k: (k, 0)),
            pl.BlockSpec((TM, NG), lambda k: (k, 0)),
            pl.BlockSpec((TM, NG), lambda k: (k, 0)),
            pl.BlockSpec((TM, 1), lambda k: (k, 0)),
            pl.BlockSpec((1, HID), lambda k: (0, 0)),
            pl.BlockSpec((HID, OUT_DIM), lambda k: (0, 0)),
            pl.BlockSpec((1, OUT_DIM), lambda k: (0, 0)),
        ],
        out_specs=pl.BlockSpec((NG, OUT_DIM), lambda k: (0, 0)),
        out_shape=jax.ShapeDtypeStruct((NG, OUT_DIM), jnp.float32),
        scratch_shapes=[
            pltpu.VMEM((NG, HID), jnp.float32),
            pltpu.VMEM((NG, HID), jnp.float32),
        ],
    )(acc0, acc1, g_pad, dinv_col, u0, u1, bt, b1r, W2, b2r)


def kernel(x, edge_index, batch, W1, b1, W2, b2):
    x = x.astype(jnp.float32)
    src = edge_index[0]
    dst = edge_index[1]
    pad_e = jnp.full((EPAD - E,), N, jnp.int32)
    srcp = jnp.concatenate([src, pad_e]).reshape(NW, KCH, CH)
    dstp = jnp.concatenate([dst, pad_e]).reshape(NW, KCH, CH)
    batchp = jnp.concatenate([batch, jnp.full((ROWS - N,), NG, jnp.int32)])
    xp = jnp.pad(x, ((0, ROWS - N), (0, 0)))
    zeros1d = jnp.zeros((ZCH,), jnp.float32)
    ones128 = jnp.ones((CH,), jnp.float32)

    deg2 = _sc_degree(dstp, ones128, zeros1d)
    deg0 = deg2[0].reshape(ROWS, 1)
    deg1 = deg2[1].reshape(ROWS, 1)

    g_pad, dinv_col = _tc_matmul(xp, W1, deg0, deg1)

    acc2, u2 = _sc_edges(srcp, dstp, g_pad, dinv_col.reshape(ROWS), batchp)

    u0 = u2[0].reshape(ROWS, NG)
    u1 = u2[1].reshape(ROWS, NG)
    pool = _tc_final(acc2[0], acc2[1], g_pad, dinv_col, u0, u1,
                     batchp.reshape(ROWS, 1), b1.reshape(1, HID), W2,
                     b2.reshape(1, OUT_DIM))
    return pool
